# baseline (speedup 1.0000x reference)
import jax
import jax.numpy as jnp
from jax.experimental import pallas as pl
from jax.experimental.pallas import tpu as pltpu

OUT_CH = 257
T = 8192
TL = 1024
GRID = T // TL
NSPLIT = 4
NBUF = 4


def _loss_body(out_hbm, tgt_hbm, res_ref, obuf, tbuf, osem, tsem, acc_ref):
    i = pl.program_id(0)
    slot = jax.lax.rem(i, NBUF)
    nxt = jax.lax.rem(i + NBUF - 1, NBUF)

    def _copies(j, s):
        cs = []
        for k in range(NSPLIT):
            c0 = k * (OUT_CH // NSPLIT)
            c1 = OUT_CH if k == NSPLIT - 1 else (k + 1) * (OUT_CH // NSPLIT)
            cs.append(pltpu.make_async_copy(
                out_hbm.at[pl.ds(c0, c1 - c0), 0, pl.ds(j * TL, TL)],
                obuf.at[s, pl.ds(c0, c1 - c0)],
                osem.at[s, k],
            ))
        cs.append(pltpu.make_async_copy(
            tgt_hbm.at[0, :, pl.ds(j * TL, TL)],
            tbuf.at[s],
            tsem.at[s],
        ))
        return cs

    def _start(j, s):
        for c in _copies(j, s):
            c.start()

    @pl.when(i == 0)
    def _prime():
        acc_ref[0] = 0.0
        acc_ref[1] = 0.0
        acc_ref[2] = 0.0
        for jj in range(NBUF - 1):
            _start(jj, jj)

    @pl.when(i + NBUF - 1 < GRID)
    def _prefetch():
        _start(i + NBUF - 1, nxt)

    for c in _copies(i, slot):
        c.wait()

    data = obuf[slot]
    y = tbuf[slot, 0:1, :]
    st = tbuf[slot, 1:2, :].astype(jnp.int32)

    m = jnp.max(data, axis=0, keepdims=True)
    s_all = jnp.sum(jnp.exp(data - m), axis=0, keepdims=True)
    x = data[0:1, :]
    s_sym = s_all - jnp.exp(x - m)
    logz = m + jnp.log(s_sym)

    ch = jax.lax.broadcasted_iota(jnp.int32, (OUT_CH, TL), 0)
    ll = jnp.sum(jnp.where(ch == st + 1, data, 0.0), axis=0, keepdims=True)
    ce = logz - ll

    bce = jnp.maximum(x, 0.0) - x * y + jnp.log1p(jnp.exp(-jnp.abs(x)))

    acc_ref[0] += jnp.sum(bce)
    acc_ref[1] += jnp.sum(ce * y)
    acc_ref[2] += jnp.sum(y)

    @pl.when(i == GRID - 1)
    def _final():
        count = acc_ref[2]
        symbol_loss = jnp.where(
            count != 0.0, acc_ref[1] / jnp.maximum(count, 1.0), 0.0
        )
        res_ref[0, 0] = acc_ref[0] / T + symbol_loss


def kernel(outputs, targets):
    ot = jnp.transpose(outputs, (2, 0, 1))
    tt = jnp.transpose(targets, (0, 2, 1))
    res = pl.pallas_call(
        _loss_body,
        grid=(GRID,),
        in_specs=[
            pl.BlockSpec(memory_space=pl.ANY),
            pl.BlockSpec(memory_space=pl.ANY),
        ],
        out_specs=pl.BlockSpec((1, 1), lambda i: (0, 0), memory_space=pltpu.SMEM),
        out_shape=jax.ShapeDtypeStruct((1, 1), jnp.float32),
        scratch_shapes=[
            pltpu.VMEM((NBUF, OUT_CH, TL), jnp.float32),
            pltpu.VMEM((NBUF, 2, TL), jnp.float32),
            pltpu.SemaphoreType.DMA((NBUF, NSPLIT)),
            pltpu.SemaphoreType.DMA((NBUF,)),
            pltpu.SMEM((3,), jnp.float32),
        ],
        compiler_params=pltpu.CompilerParams(
            dimension_semantics=("arbitrary",),
        ),
    )(ot, tt)
    return res[0, 0]


# speedup vs baseline: 1.0098x; 1.0098x over previous
import jax
import jax.numpy as jnp
from jax.experimental import pallas as pl
from jax.experimental.pallas import tpu as pltpu

OUT_CH = 257
T = 8192
TL = 2048
GRID = T // TL
NSPLIT = 4
NBUF = 3


def _loss_body(out_hbm, tgt_hbm, res_ref, obuf, tbuf, osem, tsem, acc_ref):
    i = pl.program_id(0)
    slot = jax.lax.rem(i, NBUF)
    nxt = jax.lax.rem(i + NBUF - 1, NBUF)

    def _copies(j, s):
        cs = []
        for k in range(NSPLIT):
            c0 = k * (OUT_CH // NSPLIT)
            c1 = OUT_CH if k == NSPLIT - 1 else (k + 1) * (OUT_CH // NSPLIT)
            cs.append(pltpu.make_async_copy(
                out_hbm.at[pl.ds(c0, c1 - c0), 0, pl.ds(j * TL, TL)],
                obuf.at[s, pl.ds(c0, c1 - c0)],
                osem.at[s, k],
            ))
        cs.append(pltpu.make_async_copy(
            tgt_hbm.at[0, :, pl.ds(j * TL, TL)],
            tbuf.at[s],
            tsem.at[s],
        ))
        return cs

    def _start(j, s):
        for c in _copies(j, s):
            c.start()

    @pl.when(i == 0)
    def _prime():
        acc_ref[0] = 0.0
        acc_ref[1] = 0.0
        acc_ref[2] = 0.0
        for jj in range(NBUF - 1):
            _start(jj, jj)

    @pl.when(i + NBUF - 1 < GRID)
    def _prefetch():
        _start(i + NBUF - 1, nxt)

    for c in _copies(i, slot):
        c.wait()

    data = obuf[slot]
    y = tbuf[slot, 0:1, :]
    st = tbuf[slot, 1:2, :].astype(jnp.int32)

    m = jnp.max(data, axis=0, keepdims=True)
    s_all = jnp.sum(jnp.exp(data - m), axis=0, keepdims=True)
    x = data[0:1, :]
    s_sym = s_all - jnp.exp(x - m)
    logz = m + jnp.log(s_sym)

    ch = jax.lax.broadcasted_iota(jnp.int32, (OUT_CH, TL), 0)
    ll = jnp.sum(jnp.where(ch == st + 1, data, 0.0), axis=0, keepdims=True)
    ce = logz - ll

    bce = jnp.maximum(x, 0.0) - x * y + jnp.log1p(jnp.exp(-jnp.abs(x)))

    acc_ref[0] += jnp.sum(bce)
    acc_ref[1] += jnp.sum(ce * y)
    acc_ref[2] += jnp.sum(y)

    @pl.when(i == GRID - 1)
    def _final():
        count = acc_ref[2]
        symbol_loss = jnp.where(
            count != 0.0, acc_ref[1] / jnp.maximum(count, 1.0), 0.0
        )
        res_ref[0, 0] = acc_ref[0] / T + symbol_loss


def kernel(outputs, targets):
    ot = jnp.transpose(outputs, (2, 0, 1))
    tt = jnp.transpose(targets, (0, 2, 1))
    res = pl.pallas_call(
        _loss_body,
        grid=(GRID,),
        in_specs=[
            pl.BlockSpec(memory_space=pl.ANY),
            pl.BlockSpec(memory_space=pl.ANY),
        ],
        out_specs=pl.BlockSpec((1, 1), lambda i: (0, 0), memory_space=pltpu.SMEM),
        out_shape=jax.ShapeDtypeStruct((1, 1), jnp.float32),
        scratch_shapes=[
            pltpu.VMEM((NBUF, OUT_CH, TL), jnp.float32),
            pltpu.VMEM((NBUF, 2, TL), jnp.float32),
            pltpu.SemaphoreType.DMA((NBUF, NSPLIT)),
            pltpu.SemaphoreType.DMA((NBUF,)),
            pltpu.SMEM((3,), jnp.float32),
        ],
        compiler_params=pltpu.CompilerParams(
            dimension_semantics=("arbitrary",),
        ),
    )(ot, tt)
    return res[0, 0]


# speedup vs baseline: 1.1214x; 1.1105x over previous
import jax
import jax.numpy as jnp
from jax.experimental import pallas as pl
from jax.experimental.pallas import tpu as pltpu

OUT_CH = 257
T = 8192
TL = 2048
GRID = T // TL
NSPLIT = 4
NBUF = 4


def _loss_body(out_hbm, tgt_hbm, res_ref, obuf, tbuf, osem, tsem, acc_ref):
    i = pl.program_id(0)
    slot = jax.lax.rem(i, NBUF)
    nxt = jax.lax.rem(i + NBUF - 1, NBUF)

    def _copies(j, s):
        cs = []
        for k in range(NSPLIT):
            c0 = k * (OUT_CH // NSPLIT)
            c1 = OUT_CH if k == NSPLIT - 1 else (k + 1) * (OUT_CH // NSPLIT)
            cs.append(pltpu.make_async_copy(
                out_hbm.at[pl.ds(c0, c1 - c0), 0, pl.ds(j * TL, TL)],
                obuf.at[s, pl.ds(c0, c1 - c0)],
                osem.at[s, k],
            ))
        cs.append(pltpu.make_async_copy(
            tgt_hbm.at[0, :, pl.ds(j * TL, TL)],
            tbuf.at[s],
            tsem.at[s],
        ))
        return cs

    def _start(j, s):
        for c in _copies(j, s):
            c.start()

    @pl.when(i == 0)
    def _prime():
        acc_ref[0] = 0.0
        acc_ref[1] = 0.0
        acc_ref[2] = 0.0
        for jj in range(NBUF - 1):
            _start(jj, jj)

    @pl.when(i + NBUF - 1 < GRID)
    def _prefetch():
        _start(i + NBUF - 1, nxt)

    for c in _copies(i, slot):
        c.wait()

    data = obuf[slot]
    y = tbuf[slot, 0:1, :]
    st = tbuf[slot, 1:2, :].astype(jnp.int32)

    m = jnp.max(data, axis=0, keepdims=True)
    e = jnp.exp(data - m)
    ones = jnp.ones((1, OUT_CH), dtype=jnp.float32)
    s_all = jax.lax.dot_general(
        ones, e, (((1,), (0,)), ((), ())),
        preferred_element_type=jnp.float32,
    )
    x = data[0:1, :]
    s_sym = s_all - jnp.exp(x - m)
    logz = m + jnp.log(s_sym)

    ch = jax.lax.broadcasted_iota(jnp.int32, (OUT_CH, TL), 0)
    ll = jnp.sum(jnp.where(ch == st + 1, data, 0.0), axis=0, keepdims=True)
    ce = logz - ll

    bce = jnp.maximum(x, 0.0) - x * y + jnp.log1p(jnp.exp(-jnp.abs(x)))

    acc_ref[0] += jnp.sum(bce)
    acc_ref[1] += jnp.sum(ce * y)
    acc_ref[2] += jnp.sum(y)

    @pl.when(i == GRID - 1)
    def _final():
        count = acc_ref[2]
        symbol_loss = jnp.where(
            count != 0.0, acc_ref[1] / jnp.maximum(count, 1.0), 0.0
        )
        res_ref[0, 0] = acc_ref[0] / T + symbol_loss


def kernel(outputs, targets):
    ot = jnp.transpose(outputs, (2, 0, 1))
    tt = jnp.transpose(targets, (0, 2, 1))
    res = pl.pallas_call(
        _loss_body,
        grid=(GRID,),
        in_specs=[
            pl.BlockSpec(memory_space=pl.ANY),
            pl.BlockSpec(memory_space=pl.ANY),
        ],
        out_specs=pl.BlockSpec((1, 1), lambda i: (0, 0), memory_space=pltpu.SMEM),
        out_shape=jax.ShapeDtypeStruct((1, 1), jnp.float32),
        scratch_shapes=[
            pltpu.VMEM((NBUF, OUT_CH, TL), jnp.float32),
            pltpu.VMEM((NBUF, 2, TL), jnp.float32),
            pltpu.SemaphoreType.DMA((NBUF, NSPLIT)),
            pltpu.SemaphoreType.DMA((NBUF,)),
            pltpu.SMEM((3,), jnp.float32),
        ],
        compiler_params=pltpu.CompilerParams(
            dimension_semantics=("arbitrary",),
        ),
    )(ot, tt)
    return res[0, 0]


# speedup vs baseline: 1.1863x; 1.0579x over previous
import jax
import jax.numpy as jnp
from jax.experimental import pallas as pl
from jax.experimental.pallas import tpu as pltpu

OUT_CH = 257
T = 8192
TL = 2048
GRID = T // TL
NSPLIT = 4
NBUF = 4


def _loss_body(out_hbm, tgt_hbm, res_ref, obuf, tbuf, osem, tsem, vacc_ref):
    i = pl.program_id(0)
    slot = jax.lax.rem(i, NBUF)
    nxt = jax.lax.rem(i + NBUF - 1, NBUF)

    def _copies(j, s):
        cs = []
        for k in range(NSPLIT):
            c0 = k * (OUT_CH // NSPLIT)
            c1 = OUT_CH if k == NSPLIT - 1 else (k + 1) * (OUT_CH // NSPLIT)
            cs.append(pltpu.make_async_copy(
                out_hbm.at[pl.ds(c0, c1 - c0), 0, pl.ds(j * TL, TL)],
                obuf.at[s, pl.ds(c0, c1 - c0)],
                osem.at[s, k],
            ))
        cs.append(pltpu.make_async_copy(
            tgt_hbm.at[0, :, pl.ds(j * TL, TL)],
            tbuf.at[s],
            tsem.at[s],
        ))
        return cs

    def _start(j, s):
        for c in _copies(j, s):
            c.start()

    @pl.when(i == 0)
    def _prime():
        for jj in range(NBUF - 1):
            _start(jj, jj)

    @pl.when(i + NBUF - 1 < GRID)
    def _prefetch():
        _start(i + NBUF - 1, nxt)

    for c in _copies(i, slot):
        c.wait()

    data = obuf[slot]
    y = tbuf[slot, 0:1, :]
    st = tbuf[slot, 1:2, :].astype(jnp.int32)

    m = jnp.max(data, axis=0, keepdims=True)
    e = jnp.exp(data - m)
    ones = jnp.ones((1, OUT_CH), dtype=jnp.float32)
    s_all = jax.lax.dot_general(
        ones, e, (((1,), (0,)), ((), ())),
        preferred_element_type=jnp.float32,
    )
    x = data[0:1, :]
    s_sym = s_all - jnp.exp(x - m)
    logz = m + jnp.log(s_sym)

    ch = jax.lax.broadcasted_iota(jnp.int32, (OUT_CH, TL), 0)
    ll = jax.lax.dot_general(
        ones, jnp.where(ch == st + 1, data, 0.0), (((1,), (0,)), ((), ())),
        preferred_element_type=jnp.float32,
    )
    ce = logz - ll

    bce = jnp.maximum(x, 0.0) - x * y + jnp.log1p(jnp.exp(-jnp.abs(x)))

    @pl.when(i == 0)
    def _zero_vacc():
        vacc_ref[...] = jnp.zeros((3, TL), jnp.float32)

    vacc_ref[0:1, :] += bce
    vacc_ref[1:2, :] += ce * y
    vacc_ref[2:3, :] += y

    @pl.when(i == GRID - 1)
    def _final():
        count = jnp.sum(vacc_ref[2, :])
        symbol_loss = jnp.where(
            count != 0.0,
            jnp.sum(vacc_ref[1, :]) / jnp.maximum(count, 1.0),
            0.0,
        )
        res_ref[0, 0] = jnp.sum(vacc_ref[0, :]) / T + symbol_loss


def kernel(outputs, targets):
    ot = jnp.transpose(outputs, (2, 0, 1))
    tt = jnp.transpose(targets, (0, 2, 1))
    res = pl.pallas_call(
        _loss_body,
        grid=(GRID,),
        in_specs=[
            pl.BlockSpec(memory_space=pl.ANY),
            pl.BlockSpec(memory_space=pl.ANY),
        ],
        out_specs=pl.BlockSpec((1, 1), lambda i: (0, 0), memory_space=pltpu.SMEM),
        out_shape=jax.ShapeDtypeStruct((1, 1), jnp.float32),
        scratch_shapes=[
            pltpu.VMEM((NBUF, OUT_CH, TL), jnp.float32),
            pltpu.VMEM((NBUF, 2, TL), jnp.float32),
            pltpu.SemaphoreType.DMA((NBUF, NSPLIT)),
            pltpu.SemaphoreType.DMA((NBUF,)),
            pltpu.VMEM((3, TL), jnp.float32),
        ],
        compiler_params=pltpu.CompilerParams(
            dimension_semantics=("arbitrary",),
        ),
    )(ot, tt)
    return res[0, 0]
